# baseline (speedup 1.0000x reference)
import jax
import jax.numpy as jnp
from jax import lax
from jax.experimental import pallas as pl
from jax.experimental.pallas import tpu as pltpu
from jax.experimental.pallas import tpu_sc as plsc

B = 16384
TD = 32
NNUM = 8
VD = 128
NREP = 3
QED = 32

NUM_TABLES = 5
NW = 32
BPW = B // NW
SUB = 2
BSUB = BPW // SUB


def _gather_kernel(qt, ca_t, cb_t, cc_t, cd_t,
                   qid, ca, cb, cc, cd,
                   out_q, out_a, out_b, out_c, out_d,
                   i0, i1, i2, i3, i4, r0, r1, g0, g1, w0, w1):
  nc = 2
  wid = lax.axis_index("s") * nc + lax.axis_index("c")
  base = wid * BPW

  tables = (qt, ca_t, cb_t, cc_t, cd_t)
  idxs = (qid, ca, cb, cc, cd)
  outs = (out_q, out_a, out_b, out_c, out_d)
  idx_v = (i0, i1, i2, i3, i4)
  rows = (r0, r1)
  gsem = (g0, g1)
  wsem = (w0, w1)

  for f in range(NUM_TABLES):
    pltpu.sync_copy(idxs[f].at[pl.ds(base, BPW)], idx_v[f])

  tasks = [(f, c) for f in range(NUM_TABLES) for c in range(SUB)]
  n = len(tasks)
  gathers = [None] * n
  writes = [None] * n
  for t, (f, c) in enumerate(tasks):
    b = t % 2
    if t >= 2:
      writes[t - 2].wait()
    gathers[t] = pltpu.async_copy(
        tables[f].at[idx_v[f].at[pl.ds(c * BSUB, BSUB)]], rows[b], gsem[b])
    if t >= 1:
      pf, pc = tasks[t - 1]
      gathers[t - 1].wait()
      writes[t - 1] = pltpu.async_copy(
          rows[(t - 1) % 2],
          outs[pf].at[pl.ds(base + pc * BSUB, BSUB)],
          wsem[(t - 1) % 2])
  gathers[n - 1].wait()
  lf, lc = tasks[n - 1]
  writes[n - 1] = pltpu.async_copy(
      rows[(n - 1) % 2], outs[lf].at[pl.ds(base + lc * BSUB, BSUB)],
      wsem[(n - 1) % 2])
  writes[n - 2].wait()
  writes[n - 1].wait()


def _sc_gather(qt4, ca4, cb4, cc4, cd4, qid4, ca_i4, cb_i4, cc_i4, cd_i4):
  mesh = plsc.VectorSubcoreMesh(core_axis_name="c", subcore_axis_name="s")
  out_t = tuple(
      jax.ShapeDtypeStruct((B, 128), jnp.float32) for _ in range(NUM_TABLES))
  fn = pl.kernel(
      _gather_kernel,
      out_type=out_t,
      mesh=mesh,
      scratch_types=(
          [pltpu.VMEM((BPW,), jnp.int32) for _ in range(NUM_TABLES)]
          + [pltpu.VMEM((BSUB, 128), jnp.float32) for _ in range(2)]
          + [pltpu.SemaphoreType.DMA for _ in range(4)]),
  )
  return fn(qt4, ca4, cb4, cc4, cd4, qid4, ca_i4, cb_i4, cc_i4, cd_i4)


def _extract(x128, rem):
  out = jnp.where(rem == 0, x128[:, 0:TD], 0.0)
  for p in range(1, 4):
    out = out + jnp.where(rem == p, x128[:, p * TD:(p + 1) * TD], 0.0)
  return out


def _mlp_kernel(qa, qb, qc, qd, qq, ra, rb, rc, rd, rq, num, vec,
                nw1, nb1, nw2, nb2, vw, vb, mw1, mb1, mw2, mb2,
                out):
  ea = _extract(qa[...], ra[...])
  eb = _extract(qb[...], rb[...])
  ec = _extract(qc[...], rc[...])
  ed = _extract(qd[...], rd[...])
  eq = _extract(qq[...], rq[...])
  h = jnp.maximum(
      jnp.dot(num[...], nw1[...], preferred_element_type=jnp.float32)
      + nb1[...], 0.0)
  h = jnp.dot(h, nw2[...], preferred_element_type=jnp.float32) + nb2[...]
  v = jnp.dot(vec[...], vw[...], preferred_element_type=jnp.float32) + vb[...]
  feat = jnp.concatenate([ea, eb, ec, ed, eq, h, v], axis=1)
  g = jnp.maximum(
      jnp.dot(feat, mw1[...], preferred_element_type=jnp.float32) + mb1[...],
      0.0)
  out[...] = (
      jnp.dot(g, mw2[...], preferred_element_type=jnp.float32) + mb2[...])


def _tc_mlp(packed, rems, numericals, vec_emb,
            num_W1, num_b1, num_W2, num_b2, vec_W, vec_b,
            merge_W1, merge_b1, merge_W2, merge_b2):
  BB = 2048
  grid = (B // BB,)

  def batch_spec(width):
    return pl.BlockSpec((BB, width), lambda i: (i, 0))

  def full_spec(shape):
    return pl.BlockSpec(shape, lambda i: tuple(0 for _ in shape))

  return pl.pallas_call(
      _mlp_kernel,
      grid=grid,
      in_specs=(
          [batch_spec(128) for _ in range(NUM_TABLES)]
          + [batch_spec(1) for _ in range(NUM_TABLES)]
          + [batch_spec(NNUM), batch_spec(VD),
             full_spec(num_W1.shape), full_spec(num_b1.shape),
             full_spec(num_W2.shape), full_spec(num_b2.shape),
             full_spec(vec_W.shape), full_spec(vec_b.shape),
             full_spec(merge_W1.shape), full_spec(merge_b1.shape),
             full_spec(merge_W2.shape), full_spec(merge_b2.shape)]),
      out_specs=batch_spec(QED),
      out_shape=jax.ShapeDtypeStruct((B, QED), jnp.float32),
  )(*packed, *rems, numericals, vec_emb,
    num_W1, num_b1, num_W2, num_b2, vec_W, vec_b,
    merge_W1, merge_b1, merge_W2, merge_b2)


def kernel(query_id, cat_a, cat_b, cat_c, cat_d, numericals, vec_emb,
           query_table, ct_a, ct_b, ct_c, ct_d,
           num_W1, num_b1, num_W2, num_b2,
           vec_W, vec_b,
           merge_W1, merge_b1, merge_W2, merge_b2):
  ids = [x.astype(jnp.int32)
         for x in (cat_a, cat_b, cat_c, cat_d, query_id)]
  tables = [t.reshape(t.shape[0] // 4, 128)
            for t in (ct_a, ct_b, ct_c, ct_d, query_table)]
  packed_idx = [x >> 2 for x in ids]
  rems = [(x & 3).reshape(B, 1) for x in ids]

  pa, pb, pc_, pd, pq = _sc_gather(*tables, *packed_idx)

  return _tc_mlp(
      (pa, pb, pc_, pd, pq), rems, numericals, vec_emb,
      num_W1, num_b1.reshape(1, -1), num_W2, num_b2.reshape(1, -1),
      vec_W, vec_b.reshape(1, -1),
      merge_W1, merge_b1.reshape(1, -1), merge_W2, merge_b2.reshape(1, -1))


# speedup vs baseline: 1.5783x; 1.5783x over previous
import jax
import jax.numpy as jnp
from jax import lax
from jax.experimental import pallas as pl
from jax.experimental.pallas import tpu as pltpu
from jax.experimental.pallas import tpu_sc as plsc

B = 16384
TD = 32
NNUM = 8
VD = 128
NREP = 3
QED = 32

NUM_TABLES = 5
NW = 32
BPW = B // NW
NSEM = 8


def _gather_kernel(qt, ca_t, cb_t, cc_t, cd_t,
                   qid, ca, cb, cc, cd,
                   out_q, out_a, out_b, out_c, out_d,
                   idx_hv, rows_v, osem, *gsems):
  nc = 2
  wid = lax.axis_index("s") * nc + lax.axis_index("c")
  base = wid * BPW

  tables = (qt, ca_t, cb_t, cc_t, cd_t)
  idxs = (qid, ca, cb, cc, cd)
  outs = (out_q, out_a, out_b, out_c, out_d)

  for f in range(NUM_TABLES):
    pltpu.sync_copy(idxs[f].at[pl.ds(base, BPW)], idx_hv)
    table = tables[f]

    @pl.loop(0, BPW, step=16)
    def _rows(i):
      v = idx_hv[pl.ds(i, 16)]
      for j in range(16):
        pltpu.async_copy(
            table.at[pl.ds(v[j], 1)], rows_v.at[pl.ds(i + j, 1)], gsems[0])

    pltpu.make_async_copy(table.at[pl.ds(0, BPW)], rows_v, gsems[0]).wait()
    pltpu.sync_copy(rows_v, outs[f].at[pl.ds(base, BPW)])


def _sc_gather(qt, ca_t, cb_t, cc_t, cd_t, qid, ca, cb, cc, cd):
  mesh = plsc.VectorSubcoreMesh(core_axis_name="c", subcore_axis_name="s")
  out_t = tuple(
      jax.ShapeDtypeStruct((B, TD), jnp.float32) for _ in range(NUM_TABLES))
  fn = pl.kernel(
      _gather_kernel,
      out_type=out_t,
      mesh=mesh,
      scratch_types=(
          [pltpu.VMEM((BPW,), jnp.int32),
           pltpu.VMEM((BPW, TD), jnp.float32)]
          + [pltpu.SemaphoreType.DMA for _ in range(2)]),
  )
  return fn(qt, ca_t, cb_t, cc_t, cd_t, qid, ca, cb, cc, cd)


def _mlp_kernel(ea, eb, ec, ed, eq, num, vec,
                nw1, nb1, nw2, nb2, vw, vb, mw1, mb1, mw2, mb2,
                out):
  h = jnp.maximum(
      jnp.dot(num[...], nw1[...], preferred_element_type=jnp.float32)
      + nb1[...], 0.0)
  h = jnp.dot(h, nw2[...], preferred_element_type=jnp.float32) + nb2[...]
  v = jnp.dot(vec[...], vw[...], preferred_element_type=jnp.float32) + vb[...]
  feat = jnp.concatenate(
      [ea[...], eb[...], ec[...], ed[...], eq[...], h, v], axis=1)
  g = jnp.maximum(
      jnp.dot(feat, mw1[...], preferred_element_type=jnp.float32) + mb1[...],
      0.0)
  out[...] = (
      jnp.dot(g, mw2[...], preferred_element_type=jnp.float32) + mb2[...])


def _tc_mlp(emb_a, emb_b, emb_c, emb_d, emb_q, numericals, vec_emb,
            num_W1, num_b1, num_W2, num_b2, vec_W, vec_b,
            merge_W1, merge_b1, merge_W2, merge_b2):
  BB = 2048
  grid = (B // BB,)

  def batch_spec(width):
    return pl.BlockSpec((BB, width), lambda i: (i, 0))

  def full_spec(shape):
    return pl.BlockSpec(shape, lambda i: tuple(0 for _ in shape))

  return pl.pallas_call(
      _mlp_kernel,
      grid=grid,
      in_specs=[
          batch_spec(TD), batch_spec(TD), batch_spec(TD), batch_spec(TD),
          batch_spec(TD), batch_spec(NNUM), batch_spec(VD),
          full_spec(num_W1.shape), full_spec(num_b1.shape),
          full_spec(num_W2.shape), full_spec(num_b2.shape),
          full_spec(vec_W.shape), full_spec(vec_b.shape),
          full_spec(merge_W1.shape), full_spec(merge_b1.shape),
          full_spec(merge_W2.shape), full_spec(merge_b2.shape),
      ],
      out_specs=batch_spec(QED),
      out_shape=jax.ShapeDtypeStruct((B, QED), jnp.float32),
  )(emb_a, emb_b, emb_c, emb_d, emb_q, numericals, vec_emb,
    num_W1, num_b1, num_W2, num_b2, vec_W, vec_b,
    merge_W1, merge_b1, merge_W2, merge_b2)


def kernel(query_id, cat_a, cat_b, cat_c, cat_d, numericals, vec_emb,
           query_table, ct_a, ct_b, ct_c, ct_d,
           num_W1, num_b1, num_W2, num_b2,
           vec_W, vec_b,
           merge_W1, merge_b1, merge_W2, merge_b2):
  qid = query_id.astype(jnp.int32)
  ca = cat_a.astype(jnp.int32)
  cb = cat_b.astype(jnp.int32)
  cc = cat_c.astype(jnp.int32)
  cd = cat_d.astype(jnp.int32)

  eq, ea, eb, ec, ed = _sc_gather(
      query_table, ct_a, ct_b, ct_c, ct_d, qid, ca, cb, cc, cd)

  return _tc_mlp(
      ea, eb, ec, ed, eq, numericals, vec_emb,
      num_W1, num_b1.reshape(1, -1), num_W2, num_b2.reshape(1, -1),
      vec_W, vec_b.reshape(1, -1),
      merge_W1, merge_b1.reshape(1, -1), merge_W2, merge_b2.reshape(1, -1))
